# speedup vs baseline: 1.8225x; 1.8225x over previous
import jax
import jax.numpy as jnp
from jax.experimental import pallas as pl
from jax.experimental.pallas import tpu as pltpu

_B, _D, _A, _S = 16384, 128, 64, 64
_BT = 1024


def _fused_body(state_ref, w_ref, b_ref, wv_ref, bv_ref,
                mean_ref, std_ref, value_ref, onehot_ref):
    x = state_ref[...]
    w = w_ref[...]
    y = jax.lax.dot_general(x, w, (((1,), (0,)), ((), ())),
                            preferred_element_type=jnp.float32)
    y = y + b_ref[...]
    mean_ref[...] = y[:, :_A]
    std_ref[...] = jnp.zeros((_BT, _A), jnp.float32)
    logits = y[:, _A:]
    idx = jnp.argmax(logits, axis=-1)
    iota = jax.lax.broadcasted_iota(jnp.int32, (_BT, _S), 1)
    onehot_ref[...] = (iota == idx[:, None]).astype(jnp.float32)
    value_ref[...] = jnp.sum(x * wv_ref[...], axis=1) + bv_ref[0]


def kernel(state, W_skill, b_skill, W_action, b_action, W_value, b_value):
    w_cat = jnp.concatenate([W_action, W_skill], axis=0).T
    b_cat = jnp.concatenate([b_action, b_skill]).reshape(1, _A + _S)

    grid = (_B // _BT,)
    mean, std, value, one_hot = pl.pallas_call(
        _fused_body,
        grid=grid,
        in_specs=[
            pl.BlockSpec((_BT, _D), lambda i: (i, 0)),
            pl.BlockSpec((_D, _A + _S), lambda i: (0, 0)),
            pl.BlockSpec((1, _A + _S), lambda i: (0, 0)),
            pl.BlockSpec((1, _D), lambda i: (0, 0)),
            pl.BlockSpec((1,), lambda i: (0,)),
        ],
        out_specs=[
            pl.BlockSpec((_BT, _A), lambda i: (i, 0)),
            pl.BlockSpec((_BT, _A), lambda i: (i, 0)),
            pl.BlockSpec((_BT,), lambda i: (i,)),
            pl.BlockSpec((_BT, _S), lambda i: (i, 0)),
        ],
        out_shape=[
            jax.ShapeDtypeStruct((_B, _A), jnp.float32),
            jax.ShapeDtypeStruct((_B, _A), jnp.float32),
            jax.ShapeDtypeStruct((_B,), jnp.float32),
            jax.ShapeDtypeStruct((_B, _S), jnp.float32),
        ],
        compiler_params=pltpu.CompilerParams(
            dimension_semantics=("arbitrary",),
        ),
    )(state, w_cat, b_cat, W_value, b_value)
    return (mean, std, value, one_hot)
